# baseline (speedup 1.0000x reference)
import functools

import jax
import jax.numpy as jnp
from jax import lax
from jax.experimental import pallas as pl
from jax.experimental.pallas import tpu as pltpu
from jax.experimental.pallas import tpu_sc as plsc

_GROUPS = 4
_N_EMBED = 1024
_DSUB = 64


_TBLK = 512


def _score_kernel(z_ref, embr_ref, w2r_ref, ind_ref):
    zb = z_ref[...]
    for g in range(_GROUPS):
        a = zb[:, g * _DSUB:(g + 1) * _DSUB]
        w = embr_ref[g * _N_EMBED:(g + 1) * _N_EMBED, :]
        s = lax.dot_general(
            a, w, dimension_numbers=(((1,), (1,)), ((), ())),
            preferred_element_type=jnp.float32,
        )
        f2 = jnp.sum(a * a, axis=1, keepdims=True)
        w2 = w2r_ref[g, :][None, :]
        dist = (f2 - 2.0 * s) + w2
        idx = jnp.argmin(dist, axis=1).astype(jnp.int32)
        ind_ref[g, :] = (_N_EMBED - 1) - idx


def _compute_indices(z2, emb_rev, w2_rev):
    nt = z2.shape[0]
    nb = nt // _TBLK
    return pl.pallas_call(
        _score_kernel,
        grid=(nb,),
        in_specs=[
            pl.BlockSpec((_TBLK, _GROUPS * _DSUB), lambda i: (i, 0)),
            pl.BlockSpec((_GROUPS * _N_EMBED, _DSUB), lambda i: (0, 0)),
            pl.BlockSpec((_GROUPS, _N_EMBED), lambda i: (0, 0)),
        ],
        out_specs=pl.BlockSpec((_GROUPS, _TBLK), lambda i: (0, i)),
        out_shape=jax.ShapeDtypeStruct((_GROUPS, nt), jnp.int32),
    )(z2, emb_rev, w2_rev)


_NC, _NS = 2, 16
_NW = _NC * _NS
_CHUNK = 128


def _make_gather(total_rows, d_model):
    rows_per_w = total_rows // _NW
    nchunk = rows_per_w // _CHUNK
    zrows = _CHUNK * _DSUB // d_model
    per_z = d_model // _DSUB
    mesh = plsc.VectorSubcoreMesh(core_axis_name="c", subcore_axis_name="s")

    @functools.partial(
        pl.kernel,
        mesh=mesh,
        out_type=[
            jax.ShapeDtypeStruct((total_rows // per_z, d_model), jnp.float32),
            jax.ShapeDtypeStruct((_NW, 128), jnp.float32),
        ],
        scratch_types=[
            pltpu.VMEM((nchunk, _CHUNK), jnp.int32),
            pltpu.VMEM((_CHUNK, 128), jnp.float32),
            pltpu.VMEM((zrows, d_model), jnp.float32),
            pltpu.VMEM((zrows, d_model), jnp.float32),
            pltpu.VMEM((128,), jnp.float32),
            pltpu.SemaphoreType.DMA,
        ],
    )
    def k(emb_hbm, idx_hbm, z_hbm, zq_hbm, psum_hbm, idx_v, rows_v, z_v,
          zqc_v, acc_v, sem):
        wid = lax.axis_index("s") * _NC + lax.axis_index("c")
        zbase = wid * (rows_per_w // per_z)
        pltpu.sync_copy(idx_hbm.at[wid], idx_v)
        acc = jnp.zeros((16,), jnp.float32)
        for j in range(nchunk):
            zrow0 = zbase + j * zrows
            gat = pltpu.async_copy(emb_hbm.at[idx_v.at[j]], rows_v, sem)
            pltpu.sync_copy(z_hbm.at[pl.ds(zrow0, zrows)], z_v)
            gat.wait()

            def body(zr, a):
                for q in range(d_model // 16):
                    fr = zr * per_z + q // (_DSUB // 16)
                    c = q % (_DSUB // 16)
                    g = rows_v[fr, pl.ds(c * 16, 16)]
                    d = g - z_v[zr, pl.ds(q * 16, 16)]
                    zqc_v[zr, pl.ds(q * 16, 16)] = g
                    a = a + d * d
                return a

            acc = lax.fori_loop(0, zrows, body, acc)
            pltpu.sync_copy(zqc_v, zq_hbm.at[pl.ds(zrow0, zrows)])
        for t in range(8):
            acc_v[pl.ds(t * 16, 16)] = jnp.zeros((16,), jnp.float32)
        acc_v[pl.ds(0, 16)] = acc
        pltpu.sync_copy(acc_v, psum_hbm.at[wid])

    return k


def kernel(z, embed_w):
    B, N, D = z.shape
    nt = B * N
    total = nt * _GROUPS
    z2 = z.reshape(nt, D)

    emb_rev = embed_w.reshape(_GROUPS, _N_EMBED, _DSUB)[:, ::-1]
    w2_rev = jnp.sum(emb_rev * emb_rev, axis=2)
    ind = _compute_indices(z2, emb_rev.reshape(_GROUPS * _N_EMBED, _DSUB),
                           w2_rev)
    ind_flat = ind.reshape(total)

    idx3 = ind_flat.reshape(_NW, total // _NW // _CHUNK, _CHUNK)
    emb_pad = jnp.pad(embed_w[:_N_EMBED], ((0, 0), (0, 128 - _DSUB)))
    zq2, psum = _make_gather(total, D)(emb_pad, idx3, z2)

    diff = (12.5 / (total * _DSUB)) * jnp.sum(psum[:, :16])
    z_q = zq2.reshape(B, N, D)
    ind_out = ind_flat.reshape(N, B, _GROUPS)
    return (z_q, diff, ind_out)


# speedup vs baseline: 1.4751x; 1.4751x over previous
import functools

import jax
import jax.numpy as jnp
from jax import lax
from jax.experimental import pallas as pl
from jax.experimental.pallas import tpu as pltpu
from jax.experimental.pallas import tpu_sc as plsc

_GROUPS = 4
_N_EMBED = 1024
_DSUB = 64


_TBLK = 512


def _score_kernel(z_ref, emb_ref, w2_ref, ind_ref):
    zb = z_ref[...]
    iota = lax.broadcasted_iota(
        jnp.int32, (_TBLK, _N_EMBED), 1).astype(jnp.float32)
    for g in range(_GROUPS):
        a = zb[:, g * _DSUB:(g + 1) * _DSUB]
        w = emb_ref[g * _N_EMBED:(g + 1) * _N_EMBED, :]
        s = lax.dot_general(
            a, w, dimension_numbers=(((1,), (1,)), ((), ())),
            preferred_element_type=jnp.float32,
        )
        f2 = jnp.sum(a * a, axis=1, keepdims=True)
        w2 = w2_ref[g, :][None, :]
        dist = (f2 - 2.0 * s) + w2
        m = jnp.min(dist, axis=1, keepdims=True)
        sel = jnp.where(dist == m, iota, jnp.float32(_N_EMBED))
        ind_ref[g, :] = jnp.min(sel, axis=1).astype(jnp.int32)


def _compute_indices(z2, emb_rev, w2_rev):
    nt = z2.shape[0]
    nb = nt // _TBLK
    return pl.pallas_call(
        _score_kernel,
        grid=(nb,),
        in_specs=[
            pl.BlockSpec((_TBLK, _GROUPS * _DSUB), lambda i: (i, 0)),
            pl.BlockSpec((_GROUPS * _N_EMBED, _DSUB), lambda i: (0, 0)),
            pl.BlockSpec((_GROUPS, _N_EMBED), lambda i: (0, 0)),
        ],
        out_specs=pl.BlockSpec((_GROUPS, _TBLK), lambda i: (0, i)),
        out_shape=jax.ShapeDtypeStruct((_GROUPS, nt), jnp.int32),
    )(z2, emb_rev, w2_rev)


_NC, _NS = 2, 16
_NW = _NC * _NS
_CHUNK = 128


def _make_gather(total_rows, d_model):
    rows_per_w = total_rows // _NW
    nchunk = rows_per_w // _CHUNK
    zrows = _CHUNK * _DSUB // d_model
    per_z = d_model // _DSUB
    mesh = plsc.VectorSubcoreMesh(core_axis_name="c", subcore_axis_name="s")

    @functools.partial(
        pl.kernel,
        mesh=mesh,
        out_type=[
            jax.ShapeDtypeStruct((total_rows // per_z, d_model), jnp.float32),
            jax.ShapeDtypeStruct((_NW, 128), jnp.float32),
        ],
        scratch_types=[
            pltpu.VMEM((nchunk, _CHUNK), jnp.int32),
            pltpu.VMEM((_CHUNK, 128), jnp.float32),
            pltpu.VMEM((zrows, d_model), jnp.float32),
            pltpu.VMEM((zrows, d_model), jnp.float32),
            pltpu.VMEM((128,), jnp.float32),
            pltpu.SemaphoreType.DMA,
        ],
    )
    def k(emb_hbm, idx_hbm, z_hbm, zq_hbm, psum_hbm, idx_v, rows_v, z_v,
          zqc_v, acc_v, sem):
        wid = lax.axis_index("s") * _NC + lax.axis_index("c")
        zbase = wid * (rows_per_w // per_z)
        pltpu.sync_copy(idx_hbm.at[wid], idx_v)
        acc = jnp.zeros((16,), jnp.float32)
        for j in range(nchunk):
            zrow0 = zbase + j * zrows
            gat = pltpu.async_copy(emb_hbm.at[idx_v.at[j]], rows_v, sem)
            pltpu.sync_copy(z_hbm.at[pl.ds(zrow0, zrows)], z_v)
            gat.wait()

            def body(zr, a):
                for q in range(d_model // 16):
                    fr = zr * per_z + q // (_DSUB // 16)
                    c = q % (_DSUB // 16)
                    g = rows_v[fr, pl.ds(c * 16, 16)]
                    d = g - z_v[zr, pl.ds(q * 16, 16)]
                    zqc_v[zr, pl.ds(q * 16, 16)] = g
                    a = a + d * d
                return a

            acc = lax.fori_loop(0, zrows, body, acc)
            pltpu.sync_copy(zqc_v, zq_hbm.at[pl.ds(zrow0, zrows)])
        for t in range(8):
            acc_v[pl.ds(t * 16, 16)] = jnp.zeros((16,), jnp.float32)
        acc_v[pl.ds(0, 16)] = acc
        pltpu.sync_copy(acc_v, psum_hbm.at[wid])

    return k


def kernel(z, embed_w):
    B, N, D = z.shape
    nt = B * N
    total = nt * _GROUPS
    z2 = z.reshape(nt, D)

    w2_all = jnp.sum(embed_w * embed_w, axis=1).reshape(_GROUPS, _N_EMBED)
    ind = _compute_indices(z2, embed_w, w2_all)
    ind_flat = ind.reshape(total)

    idx3 = ind_flat.reshape(_NW, total // _NW // _CHUNK, _CHUNK)
    emb_pad = jnp.pad(embed_w[:_N_EMBED], ((0, 0), (0, 128 - _DSUB)))
    zq2, psum = _make_gather(total, D)(emb_pad, idx3, z2)

    diff = (12.5 / (total * _DSUB)) * jnp.sum(psum[:, :16])
    z_q = zq2.reshape(B, N, D)
    ind_out = ind_flat.reshape(N, B, _GROUPS)
    return (z_q, diff, ind_out)


# speedup vs baseline: 1.6187x; 1.0974x over previous
import functools

import jax
import jax.numpy as jnp
from jax import lax
from jax.experimental import pallas as pl
from jax.experimental.pallas import tpu as pltpu
from jax.experimental.pallas import tpu_sc as plsc

_GROUPS = 4
_N_EMBED = 1024
_DSUB = 64


_TBLK = 512


def _score_kernel(z_ref, emb_ref, w2_ref, ind_ref):
    zb = z_ref[...]
    iota = lax.broadcasted_iota(
        jnp.int32, (_TBLK, _N_EMBED), 1).astype(jnp.float32)
    for g in range(_GROUPS):
        a = zb[:, g * _DSUB:(g + 1) * _DSUB]
        w = emb_ref[g * _N_EMBED:(g + 1) * _N_EMBED, :]
        s = lax.dot_general(
            a, w, dimension_numbers=(((1,), (1,)), ((), ())),
            preferred_element_type=jnp.float32,
        )
        f2 = jnp.sum(a * a, axis=1, keepdims=True)
        w2 = w2_ref[g, :][None, :]
        dist = (f2 - 2.0 * s) + w2
        m = jnp.min(dist, axis=1, keepdims=True)
        sel = jnp.where(dist == m, iota, jnp.float32(_N_EMBED))
        ind_ref[g, :] = jnp.min(sel, axis=1).astype(jnp.int32)


def _compute_indices(z2, emb_rev, w2_rev):
    nt = z2.shape[0]
    nb = nt // _TBLK
    return pl.pallas_call(
        _score_kernel,
        grid=(nb,),
        in_specs=[
            pl.BlockSpec((_TBLK, _GROUPS * _DSUB), lambda i: (i, 0)),
            pl.BlockSpec((_GROUPS * _N_EMBED, _DSUB), lambda i: (0, 0)),
            pl.BlockSpec((_GROUPS, _N_EMBED), lambda i: (0, 0)),
        ],
        out_specs=pl.BlockSpec((_GROUPS, _TBLK), lambda i: (0, i)),
        out_shape=jax.ShapeDtypeStruct((_GROUPS, nt), jnp.int32),
    )(z2, emb_rev, w2_rev)


_NC, _NS = 2, 16
_NW = _NC * _NS
_CHUNK = 128


def _make_gather(total_rows, d_model):
    rows_per_w = total_rows // _NW
    nchunk = rows_per_w // _CHUNK
    zrows = _CHUNK * _DSUB // d_model
    per_z = d_model // _DSUB
    mesh = plsc.VectorSubcoreMesh(core_axis_name="c", subcore_axis_name="s")

    @functools.partial(
        pl.kernel,
        mesh=mesh,
        out_type=[
            jax.ShapeDtypeStruct((total_rows // per_z, d_model), jnp.float32),
            jax.ShapeDtypeStruct((_NW, 128), jnp.float32),
        ],
        scratch_types=[
            pltpu.VMEM((nchunk, _CHUNK), jnp.int32),
            pltpu.VMEM((2, _CHUNK, 128), jnp.float32),
            pltpu.VMEM((2, zrows, d_model), jnp.float32),
            pltpu.VMEM((2, zrows, d_model), jnp.float32),
            pltpu.VMEM((128,), jnp.float32),
            pltpu.SemaphoreType.DMA,
            pltpu.SemaphoreType.DMA,
            pltpu.SemaphoreType.DMA,
            pltpu.SemaphoreType.DMA,
            pltpu.SemaphoreType.DMA,
            pltpu.SemaphoreType.DMA,
        ],
    )
    def k(emb_hbm, idx_hbm, z_hbm, zq_hbm, psum_hbm, idx_v, rows_v, z_v,
          zqc_v, acc_v, sg0, sg1, sz0, sz1, sw0, sw1):
        wid = lax.axis_index("s") * _NC + lax.axis_index("c")
        zbase = wid * (rows_per_w // per_z)
        pltpu.sync_copy(idx_hbm.at[wid], idx_v)
        sg = (sg0, sg1)
        sz = (sz0, sz1)
        sw = (sw0, sw1)

        def start(j):
            b = j % 2
            zrow0 = zbase + j * zrows
            g = pltpu.async_copy(emb_hbm.at[idx_v.at[j]], rows_v.at[b], sg[b])
            zc = pltpu.async_copy(z_hbm.at[pl.ds(zrow0, zrows)], z_v.at[b],
                                  sz[b])
            return g, zc

        pend = {0: start(0)}
        wout = {}
        accs = [jnp.zeros((16,), jnp.float32) for _ in range(4)]
        for j in range(nchunk):
            b = j % 2
            if j + 1 < nchunk:
                pend[j + 1] = start(j + 1)
            g, zc = pend.pop(j)
            g.wait()
            zc.wait()
            if j >= 2:
                wout.pop(j - 2).wait()

            def body(zr, a):
                a = list(a)
                for q in range(d_model // 16):
                    fr = zr * per_z + q // (_DSUB // 16)
                    c = q % (_DSUB // 16)
                    gg = rows_v[b, fr, pl.ds(c * 16, 16)]
                    d = gg - z_v[b, zr, pl.ds(q * 16, 16)]
                    zqc_v[b, zr, pl.ds(q * 16, 16)] = gg
                    a[q % 4] = a[q % 4] + d * d
                return tuple(a)

            accs = lax.fori_loop(0, zrows, body, tuple(accs))
            zrow0 = zbase + j * zrows
            wout[j] = pltpu.async_copy(
                zqc_v.at[b], zq_hbm.at[pl.ds(zrow0, zrows)], sw[b])
        for j in list(wout):
            wout.pop(j).wait()
        acc = (accs[0] + accs[1]) + (accs[2] + accs[3])
        for t in range(8):
            acc_v[pl.ds(t * 16, 16)] = jnp.zeros((16,), jnp.float32)
        acc_v[pl.ds(0, 16)] = acc
        pltpu.sync_copy(acc_v, psum_hbm.at[wid])

    return k


def kernel(z, embed_w):
    B, N, D = z.shape
    nt = B * N
    total = nt * _GROUPS
    z2 = z.reshape(nt, D)

    w2_all = jnp.sum(embed_w * embed_w, axis=1).reshape(_GROUPS, _N_EMBED)
    ind = _compute_indices(z2, embed_w, w2_all)
    ind_flat = ind.reshape(total)

    idx3 = ind_flat.reshape(_NW, total // _NW // _CHUNK, _CHUNK)
    emb_pad = jnp.pad(embed_w[:_N_EMBED], ((0, 0), (0, 128 - _DSUB)))
    zq2, psum = _make_gather(total, D)(emb_pad, idx3, z2)

    diff = (12.5 / (total * _DSUB)) * jnp.sum(psum[:, :16])
    z_q = zq2.reshape(B, N, D)
    ind_out = ind_flat.reshape(N, B, _GROUPS)
    return (z_q, diff, ind_out)


# speedup vs baseline: 1.7573x; 1.0856x over previous
import functools

import jax
import jax.numpy as jnp
from jax import lax
from jax.experimental import pallas as pl
from jax.experimental.pallas import tpu as pltpu
from jax.experimental.pallas import tpu_sc as plsc

_GROUPS = 4
_N_EMBED = 1024
_DSUB = 64


_TBLK = 1024


def _score_kernel(z_ref, emb_ref, w2_ref, ind_ref):
    zb = z_ref[...]
    iota = lax.broadcasted_iota(
        jnp.int32, (_TBLK, _N_EMBED), 1).astype(jnp.float32)
    for g in range(_GROUPS):
        a = zb[:, g * _DSUB:(g + 1) * _DSUB]
        w = emb_ref[g * _N_EMBED:(g + 1) * _N_EMBED, :]
        s = lax.dot_general(
            a, w, dimension_numbers=(((1,), (1,)), ((), ())),
            preferred_element_type=jnp.float32,
        )
        f2 = jnp.sum(a * a, axis=1, keepdims=True)
        w2 = w2_ref[g, :][None, :]
        dist = (f2 - 2.0 * s) + w2
        m = jnp.min(dist, axis=1, keepdims=True)
        sel = jnp.where(dist == m, iota, jnp.float32(_N_EMBED))
        ind_ref[g, :] = jnp.min(sel, axis=1).astype(jnp.int32)


def _compute_indices(z2, emb_rev, w2_rev):
    nt = z2.shape[0]
    nb = nt // _TBLK
    return pl.pallas_call(
        _score_kernel,
        grid=(nb,),
        in_specs=[
            pl.BlockSpec((_TBLK, _GROUPS * _DSUB), lambda i: (i, 0)),
            pl.BlockSpec((_GROUPS * _N_EMBED, _DSUB), lambda i: (0, 0)),
            pl.BlockSpec((_GROUPS, _N_EMBED), lambda i: (0, 0)),
        ],
        out_specs=pl.BlockSpec((_GROUPS, _TBLK), lambda i: (0, i)),
        out_shape=jax.ShapeDtypeStruct((_GROUPS, nt), jnp.int32),
    )(z2, emb_rev, w2_rev)


_NC, _NS = 2, 16
_NW = _NC * _NS
_CHUNK = 128


def _make_gather(total_rows, d_model):
    rows_per_w = total_rows // _NW
    nchunk = rows_per_w // _CHUNK
    zrows = _CHUNK * _DSUB // d_model
    per_z = d_model // _DSUB
    mesh = plsc.VectorSubcoreMesh(core_axis_name="c", subcore_axis_name="s")

    @functools.partial(
        pl.kernel,
        mesh=mesh,
        out_type=[
            jax.ShapeDtypeStruct((total_rows // per_z, d_model), jnp.float32),
            jax.ShapeDtypeStruct((_NW, 128), jnp.float32),
        ],
        scratch_types=[
            pltpu.VMEM((nchunk, _CHUNK), jnp.int32),
            pltpu.VMEM((2, _CHUNK, 128), jnp.float32),
            pltpu.VMEM((2, zrows, d_model), jnp.float32),
            pltpu.VMEM((2, zrows, d_model), jnp.float32),
            pltpu.VMEM((128,), jnp.float32),
            pltpu.SemaphoreType.DMA,
            pltpu.SemaphoreType.DMA,
            pltpu.SemaphoreType.DMA,
            pltpu.SemaphoreType.DMA,
            pltpu.SemaphoreType.DMA,
            pltpu.SemaphoreType.DMA,
        ],
    )
    def k(emb_hbm, idx_hbm, z_hbm, zq_hbm, psum_hbm, idx_v, rows_v, z_v,
          zqc_v, acc_v, sg0, sg1, sz0, sz1, sw0, sw1):
        wid = lax.axis_index("s") * _NC + lax.axis_index("c")
        zbase = wid * (rows_per_w // per_z)
        pltpu.sync_copy(idx_hbm.at[wid], idx_v)
        sg = (sg0, sg1)
        sz = (sz0, sz1)
        sw = (sw0, sw1)

        def start(j):
            b = j % 2
            zrow0 = zbase + j * zrows
            g = pltpu.async_copy(emb_hbm.at[idx_v.at[j]], rows_v.at[b], sg[b])
            zc = pltpu.async_copy(z_hbm.at[pl.ds(zrow0, zrows)], z_v.at[b],
                                  sz[b])
            return g, zc

        pend = {0: start(0)}
        wout = {}
        accs = [jnp.zeros((16,), jnp.float32) for _ in range(4)]
        for j in range(nchunk):
            b = j % 2
            if j + 1 < nchunk:
                pend[j + 1] = start(j + 1)
            g, zc = pend.pop(j)
            g.wait()
            zc.wait()
            if j >= 2:
                wout.pop(j - 2).wait()

            def body(zr, a):
                a = list(a)
                for q in range(d_model // 16):
                    fr = zr * per_z + q // (_DSUB // 16)
                    c = q % (_DSUB // 16)
                    gg = rows_v[b, fr, pl.ds(c * 16, 16)]
                    d = gg - z_v[b, zr, pl.ds(q * 16, 16)]
                    zqc_v[b, zr, pl.ds(q * 16, 16)] = gg
                    a[q % 4] = a[q % 4] + d * d
                return tuple(a)

            accs = lax.fori_loop(0, zrows, body, tuple(accs))
            zrow0 = zbase + j * zrows
            wout[j] = pltpu.async_copy(
                zqc_v.at[b], zq_hbm.at[pl.ds(zrow0, zrows)], sw[b])
        for j in list(wout):
            wout.pop(j).wait()
        acc = (accs[0] + accs[1]) + (accs[2] + accs[3])
        for t in range(8):
            acc_v[pl.ds(t * 16, 16)] = jnp.zeros((16,), jnp.float32)
        acc_v[pl.ds(0, 16)] = acc
        pltpu.sync_copy(acc_v, psum_hbm.at[wid])

    return k


def kernel(z, embed_w):
    B, N, D = z.shape
    nt = B * N
    total = nt * _GROUPS
    z2 = z.reshape(nt, D)

    w2_all = jnp.sum(embed_w * embed_w, axis=1).reshape(_GROUPS, _N_EMBED)
    ind = _compute_indices(z2, embed_w, w2_all)
    ind_flat = ind.reshape(total)

    idx3 = ind_flat.reshape(_NW, total // _NW // _CHUNK, _CHUNK)
    emb_pad = jnp.pad(embed_w[:_N_EMBED], ((0, 0), (0, 128 - _DSUB)))
    zq2, psum = _make_gather(total, D)(emb_pad, idx3, z2)

    diff = (12.5 / (total * _DSUB)) * jnp.sum(psum[:, :16])
    z_q = zq2.reshape(B, N, D)
    ind_out = ind_flat.reshape(N, B, _GROUPS)
    return (z_q, diff, ind_out)


# speedup vs baseline: 1.8087x; 1.0293x over previous
import functools

import jax
import jax.numpy as jnp
from jax import lax
from jax.experimental import pallas as pl
from jax.experimental.pallas import tpu as pltpu
from jax.experimental.pallas import tpu_sc as plsc

_GROUPS = 4
_N_EMBED = 1024
_DSUB = 64


_TBLK = 2304


def _score_kernel(z_ref, emb_ref, w2_ref, ind_ref):
    zb = z_ref[...]
    iota = lax.broadcasted_iota(
        jnp.int32, (_TBLK, _N_EMBED), 1).astype(jnp.float32)
    for g in range(_GROUPS):
        a = zb[:, g * _DSUB:(g + 1) * _DSUB]
        w = emb_ref[g * _N_EMBED:(g + 1) * _N_EMBED, :]
        s = lax.dot_general(
            a, w, dimension_numbers=(((1,), (1,)), ((), ())),
            preferred_element_type=jnp.float32,
        )
        f2 = jnp.sum(a * a, axis=1, keepdims=True)
        w2 = w2_ref[g, :][None, :]
        dist = (f2 - 2.0 * s) + w2
        m = jnp.min(dist, axis=1, keepdims=True)
        sel = jnp.where(dist == m, iota, jnp.float32(_N_EMBED))
        ind_ref[g, :] = jnp.min(sel, axis=1).astype(jnp.int32)


def _compute_indices(z2, emb_rev, w2_rev):
    nt = z2.shape[0]
    nb = nt // _TBLK
    return pl.pallas_call(
        _score_kernel,
        grid=(nb,),
        in_specs=[
            pl.BlockSpec((_TBLK, _GROUPS * _DSUB), lambda i: (i, 0)),
            pl.BlockSpec((_GROUPS * _N_EMBED, _DSUB), lambda i: (0, 0)),
            pl.BlockSpec((_GROUPS, _N_EMBED), lambda i: (0, 0)),
        ],
        out_specs=pl.BlockSpec((_GROUPS, _TBLK), lambda i: (0, i)),
        out_shape=jax.ShapeDtypeStruct((_GROUPS, nt), jnp.int32),
    )(z2, emb_rev, w2_rev)


_NC, _NS = 2, 16
_NW = _NC * _NS
_CHUNK = 128


def _make_gather(total_rows, d_model):
    rows_per_w = total_rows // _NW
    nchunk = rows_per_w // _CHUNK
    zrows = _CHUNK * _DSUB // d_model
    per_z = d_model // _DSUB
    mesh = plsc.VectorSubcoreMesh(core_axis_name="c", subcore_axis_name="s")

    @functools.partial(
        pl.kernel,
        mesh=mesh,
        out_type=[
            jax.ShapeDtypeStruct((total_rows // per_z, d_model), jnp.float32),
            jax.ShapeDtypeStruct((_NW, 128), jnp.float32),
        ],
        scratch_types=[
            pltpu.VMEM((nchunk, _CHUNK), jnp.int32),
            pltpu.VMEM((2, _CHUNK, 128), jnp.float32),
            pltpu.VMEM((2, zrows, d_model), jnp.float32),
            pltpu.VMEM((2, zrows, d_model), jnp.float32),
            pltpu.VMEM((128,), jnp.float32),
            pltpu.SemaphoreType.DMA,
            pltpu.SemaphoreType.DMA,
            pltpu.SemaphoreType.DMA,
            pltpu.SemaphoreType.DMA,
            pltpu.SemaphoreType.DMA,
            pltpu.SemaphoreType.DMA,
        ],
    )
    def k(emb_hbm, idx_hbm, z_hbm, zq_hbm, psum_hbm, idx_v, rows_v, z_v,
          zqc_v, acc_v, sg0, sg1, sz0, sz1, sw0, sw1):
        wid = lax.axis_index("s") * _NC + lax.axis_index("c")
        zbase = wid * (rows_per_w // per_z)
        pltpu.sync_copy(idx_hbm.at[wid], idx_v)
        sg = (sg0, sg1)
        sz = (sz0, sz1)
        sw = (sw0, sw1)

        def start(j):
            b = j % 2
            zrow0 = zbase + j * zrows
            g = pltpu.async_copy(emb_hbm.at[idx_v.at[j]], rows_v.at[b], sg[b])
            zc = pltpu.async_copy(z_hbm.at[pl.ds(zrow0, zrows)], z_v.at[b],
                                  sz[b])
            return g, zc

        pend = {0: start(0)}
        wout = {}
        accs = [jnp.zeros((16,), jnp.float32) for _ in range(4)]
        for j in range(nchunk):
            b = j % 2
            if j + 1 < nchunk:
                pend[j + 1] = start(j + 1)
            g, zc = pend.pop(j)
            g.wait()
            zc.wait()
            if j >= 2:
                wout.pop(j - 2).wait()

            def body(zr, a):
                a = list(a)
                for q in range(d_model // 16):
                    fr = zr * per_z + q // (_DSUB // 16)
                    c = q % (_DSUB // 16)
                    gg = rows_v[b, fr, pl.ds(c * 16, 16)]
                    d = gg - z_v[b, zr, pl.ds(q * 16, 16)]
                    zqc_v[b, zr, pl.ds(q * 16, 16)] = gg
                    a[q % 4] = a[q % 4] + d * d
                return tuple(a)

            accs = lax.fori_loop(0, zrows, body, tuple(accs))
            zrow0 = zbase + j * zrows
            wout[j] = pltpu.async_copy(
                zqc_v.at[b], zq_hbm.at[pl.ds(zrow0, zrows)], sw[b])
        for j in list(wout):
            wout.pop(j).wait()
        acc = (accs[0] + accs[1]) + (accs[2] + accs[3])
        for t in range(8):
            acc_v[pl.ds(t * 16, 16)] = jnp.zeros((16,), jnp.float32)
        acc_v[pl.ds(0, 16)] = acc
        pltpu.sync_copy(acc_v, psum_hbm.at[wid])

    return k


def kernel(z, embed_w):
    B, N, D = z.shape
    nt = B * N
    total = nt * _GROUPS
    z2 = z.reshape(nt, D)

    w2_all = jnp.sum(embed_w * embed_w, axis=1).reshape(_GROUPS, _N_EMBED)
    ind = _compute_indices(z2, embed_w, w2_all)
    ind_flat = ind.reshape(total)

    idx3 = ind_flat.reshape(_NW, total // _NW // _CHUNK, _CHUNK)
    emb_pad = jnp.pad(embed_w[:_N_EMBED], ((0, 0), (0, 128 - _DSUB)))
    zq2, psum = _make_gather(total, D)(emb_pad, idx3, z2)

    diff = (12.5 / (total * _DSUB)) * jnp.sum(psum[:, :16])
    z_q = zq2.reshape(B, N, D)
    ind_out = ind_flat.reshape(N, B, _GROUPS)
    return (z_q, diff, ind_out)


# speedup vs baseline: 1.8136x; 1.0027x over previous
import functools

import jax
import jax.numpy as jnp
from jax import lax
from jax.experimental import pallas as pl
from jax.experimental.pallas import tpu as pltpu
from jax.experimental.pallas import tpu_sc as plsc

_GROUPS = 4
_N_EMBED = 1024
_DSUB = 64


_TBLK = 2304


def _score_kernel(z_ref, emb_ref, w2_ref, ind_ref):
    zb = z_ref[...]
    iota = lax.broadcasted_iota(
        jnp.int32, (_TBLK, _N_EMBED), 1).astype(jnp.float32)
    for g in range(_GROUPS):
        a = zb[:, g * _DSUB:(g + 1) * _DSUB]
        w = emb_ref[g * _N_EMBED:(g + 1) * _N_EMBED, :]
        s = lax.dot_general(
            a, w, dimension_numbers=(((1,), (1,)), ((), ())),
            preferred_element_type=jnp.float32,
        )
        f2 = jnp.sum(a * a, axis=1, keepdims=True)
        w2 = w2_ref[g, :][None, :]
        dist = (f2 - 2.0 * s) + w2
        m = jnp.min(dist, axis=1, keepdims=True)
        sel = jnp.where(dist == m, iota, jnp.float32(_N_EMBED))
        ind_ref[g, :] = jnp.min(sel, axis=1).astype(jnp.int32)


def _compute_indices(z2, embed_w, w2_all):
    nt = z2.shape[0]
    nb = nt // _TBLK
    return pl.pallas_call(
        _score_kernel,
        grid=(nb,),
        in_specs=[
            pl.BlockSpec((_TBLK, _GROUPS * _DSUB), lambda i: (i, 0)),
            pl.BlockSpec((_GROUPS * _N_EMBED, _DSUB), lambda i: (0, 0)),
            pl.BlockSpec((_GROUPS, _N_EMBED), lambda i: (0, 0)),
        ],
        out_specs=pl.BlockSpec((_GROUPS, _TBLK), lambda i: (0, i)),
        out_shape=jax.ShapeDtypeStruct((_GROUPS, nt), jnp.int32),
    )(z2, embed_w, w2_all)


_NC, _NS = 2, 16
_NW = _NC * _NS
_CHUNK = 128


def _make_gather(total_rows, d_model):
    rows_per_w = total_rows // _NW
    nchunk = rows_per_w // _CHUNK
    zrows = _CHUNK * _DSUB // d_model
    per_z = d_model // _DSUB
    mesh = plsc.VectorSubcoreMesh(core_axis_name="c", subcore_axis_name="s")

    @functools.partial(
        pl.kernel,
        mesh=mesh,
        out_type=[
            jax.ShapeDtypeStruct((total_rows // per_z, d_model), jnp.float32),
            jax.ShapeDtypeStruct((_NW, 128), jnp.float32),
        ],
        scratch_types=[
            pltpu.VMEM((nchunk, _CHUNK), jnp.int32),
            pltpu.VMEM((2, _CHUNK, 128), jnp.float32),
            pltpu.VMEM((2, zrows, d_model), jnp.float32),
            pltpu.VMEM((2, zrows, d_model), jnp.float32),
            pltpu.VMEM((128,), jnp.float32),
            pltpu.SemaphoreType.DMA,
            pltpu.SemaphoreType.DMA,
            pltpu.SemaphoreType.DMA,
            pltpu.SemaphoreType.DMA,
            pltpu.SemaphoreType.DMA,
            pltpu.SemaphoreType.DMA,
        ],
    )
    def k(emb_hbm, idx_hbm, z_hbm, zq_hbm, psum_hbm, idx_v, rows_v, z_v,
          zqc_v, acc_v, sg0, sg1, sz0, sz1, sw0, sw1):
        wid = lax.axis_index("s") * _NC + lax.axis_index("c")
        zbase = wid * (rows_per_w // per_z)
        pltpu.sync_copy(idx_hbm.at[wid], idx_v)
        sg = (sg0, sg1)
        sz = (sz0, sz1)
        sw = (sw0, sw1)

        def start(j):
            b = j % 2
            zrow0 = zbase + j * zrows
            g = pltpu.async_copy(emb_hbm.at[idx_v.at[j]], rows_v.at[b], sg[b])
            zc = pltpu.async_copy(z_hbm.at[pl.ds(zrow0, zrows)], z_v.at[b],
                                  sz[b])
            return g, zc

        pend = {0: start(0)}
        wout = {}
        accs = [jnp.zeros((16,), jnp.float32) for _ in range(4)]
        for j in range(nchunk):
            b = j % 2
            if j + 1 < nchunk:
                pend[j + 1] = start(j + 1)
            g, zc = pend.pop(j)
            g.wait()
            zc.wait()
            if j >= 2:
                wout.pop(j - 2).wait()

            def body(zr, a):
                a = list(a)
                for q in range(d_model // 16):
                    fr = zr * per_z + q // (_DSUB // 16)
                    c = q % (_DSUB // 16)
                    gg = rows_v[b, fr, pl.ds(c * 16, 16)]
                    d = gg - z_v[b, zr, pl.ds(q * 16, 16)]
                    zqc_v[b, zr, pl.ds(q * 16, 16)] = gg
                    a[q % 4] = a[q % 4] + d * d
                return tuple(a)

            accs = lax.fori_loop(0, zrows, body, tuple(accs))
            zrow0 = zbase + j * zrows
            wout[j] = pltpu.async_copy(
                zqc_v.at[b], zq_hbm.at[pl.ds(zrow0, zrows)], sw[b])
        for j in list(wout):
            wout.pop(j).wait()
        acc = (accs[0] + accs[1]) + (accs[2] + accs[3])
        for t in range(8):
            acc_v[pl.ds(t * 16, 16)] = jnp.zeros((16,), jnp.float32)
        acc_v[pl.ds(0, 16)] = acc
        pltpu.sync_copy(acc_v, psum_hbm.at[wid])

    return k


def kernel(z, embed_w):
    B, N, D = z.shape
    nt = B * N
    total = nt * _GROUPS
    z2 = z.reshape(nt, D)

    w2_all = jnp.sum(embed_w * embed_w, axis=1).reshape(_GROUPS, _N_EMBED)
    ind = _compute_indices(z2, embed_w, w2_all)
    ind_flat = ind.reshape(total)

    idx3 = ind_flat.reshape(_NW, total // _NW // _CHUNK, _CHUNK)
    emb_pad = jnp.pad(embed_w[:_N_EMBED], ((0, 0), (0, 128 - _DSUB)))
    zq2, psum = _make_gather(total, D)(emb_pad, idx3, z2)

    diff = (12.5 / (total * _DSUB)) * jnp.sum(psum[:, :16])
    z_q = zq2.reshape(B, N, D)
    ind_out = ind_flat.reshape(N, B, _GROUPS)
    return (z_q, diff, ind_out)
